# Initial kernel scaffold; baseline (speedup 1.0000x reference)
import jax
import jax.numpy as jnp
from jax.experimental import pallas as pl


def kernel(tokens, masks, table, pe):
    raise NotImplementedError("write your pallas kernel here")


# speedup vs baseline: 1.2426x; 1.2426x over previous
import functools

import jax
import jax.numpy as jnp
from jax import lax
from jax.experimental import pallas as pl
from jax.experimental.pallas import tpu as pltpu
from jax.experimental.pallas import tpu_sc as plsc

D_MODEL = 128


@functools.lru_cache(maxsize=None)
def _make_gather(B, V, D):
    info = plsc.get_sparse_core_info()
    NC, NS = info.num_cores, info.num_subcores
    NW = NC * NS
    assert B % NW == 0
    b_per_w = B // NW
    chunk = 400
    assert b_per_w % chunk == 0 and chunk % 8 == 0
    n_chunks = b_per_w // chunk

    mesh = plsc.VectorSubcoreMesh(core_axis_name="c", subcore_axis_name="s")

    @functools.partial(
        pl.kernel,
        mesh=mesh,
        out_type=jax.ShapeDtypeStruct((B, D), jnp.float32),
        scratch_types=[
            pltpu.VMEM((b_per_w,), jnp.int32),
            pltpu.VMEM((chunk, D), jnp.float32),
            pltpu.SemaphoreType.DMA,
        ],
    )
    def gather_kernel(idx_hbm, table_hbm, out_hbm, idx_v, rows_v, sem):
        wid = lax.axis_index("s") * NC + lax.axis_index("c")
        base = wid * b_per_w
        pltpu.sync_copy(idx_hbm.at[pl.ds(base, b_per_w)], idx_v)

        def body(i, _):
            pltpu.async_copy(
                table_hbm.at[idx_v.at[pl.ds(i * chunk, chunk)]], rows_v, sem
            ).wait()
            pltpu.sync_copy(rows_v, out_hbm.at[pl.ds(base + i * chunk, chunk)])
            return 0

        lax.fori_loop(0, n_chunks, body, 0)

    return gather_kernel


def kernel(tokens, masks, table, pe):
    B, S = tokens.shape
    idx = tokens.reshape(-1).astype(jnp.int32)
    gather = _make_gather(B * S, table.shape[0], table.shape[1])
    x = gather(idx, table).reshape(B, S, D_MODEL)
    pos_embed = pe[:S][None, :, :]
    return (x, masks, pos_embed)


# speedup vs baseline: 1.2762x; 1.0270x over previous
import functools

import jax
import jax.numpy as jnp
from jax import lax
from jax.experimental import pallas as pl
from jax.experimental.pallas import tpu as pltpu
from jax.experimental.pallas import tpu_sc as plsc

D_MODEL = 128


@functools.lru_cache(maxsize=None)
def _make_gather(B, V, D):
    info = plsc.get_sparse_core_info()
    NC, NS = info.num_cores, info.num_subcores
    NW = NC * NS
    assert B % NW == 0
    b_per_w = B // NW
    chunk = 400
    assert b_per_w % chunk == 0 and chunk % 8 == 0
    n_chunks = b_per_w // chunk

    mesh = plsc.VectorSubcoreMesh(core_axis_name="c", subcore_axis_name="s")

    assert n_chunks % 2 == 0 and n_chunks >= 4

    @functools.partial(
        pl.kernel,
        mesh=mesh,
        out_type=jax.ShapeDtypeStruct((B, D), jnp.float32),
        scratch_types=[
            pltpu.VMEM((b_per_w,), jnp.int32),
            pltpu.VMEM((chunk, D), jnp.float32),
            pltpu.VMEM((chunk, D), jnp.float32),
            pltpu.SemaphoreType.DMA,
            pltpu.SemaphoreType.DMA,
        ],
    )
    def gather_kernel(idx_hbm, table_hbm, out_hbm, idx_v, buf0, buf1, sem0, sem1):
        wid = lax.axis_index("s") * NC + lax.axis_index("c")
        base = wid * b_per_w
        pltpu.sync_copy(idx_hbm.at[pl.ds(base, b_per_w)], idx_v)
        bufs = (buf0, buf1)
        sems = (sem0, sem1)

        def start(c, b):
            pltpu.async_copy(
                table_hbm.at[idx_v.at[pl.ds(c * chunk, chunk)]], bufs[b], sems[b]
            )

        def finish(c, b):
            pltpu.make_async_copy(
                table_hbm.at[idx_v.at[pl.ds(c * chunk, chunk)]], bufs[b], sems[b]
            ).wait()
            pltpu.sync_copy(bufs[b], out_hbm.at[pl.ds(base + c * chunk, chunk)])

        start(0, 0)
        start(1, 1)

        def body(g, _):
            for b in range(2):
                finish(g + b, b)
                start(g + b + 2, b)
            return 0

        lax.fori_loop(0, (n_chunks - 2) // 2, lambda i, c: body(i * 2, c), 0)
        for b in range(2):
            finish(n_chunks - 2 + b, b)

    return gather_kernel


def kernel(tokens, masks, table, pe):
    B, S = tokens.shape
    idx = tokens.reshape(-1).astype(jnp.int32)
    gather = _make_gather(B * S, table.shape[0], table.shape[1])
    x = gather(idx, table).reshape(B, S, D_MODEL)
    pos_embed = pe[:S][None, :, :]
    return (x, masks, pos_embed)


# speedup vs baseline: 1.2795x; 1.0026x over previous
import functools

import jax
import jax.numpy as jnp
from jax import lax
from jax.experimental import pallas as pl
from jax.experimental.pallas import tpu as pltpu
from jax.experimental.pallas import tpu_sc as plsc

D_MODEL = 128


@functools.lru_cache(maxsize=None)
def _make_gather(B, V, D):
    info = plsc.get_sparse_core_info()
    NC, NS = info.num_cores, info.num_subcores
    NW = NC * NS
    assert B % NW == 0
    b_per_w = B // NW
    chunk = 200
    depth = 4
    assert b_per_w % chunk == 0 and chunk % 8 == 0
    n_chunks = b_per_w // chunk
    assert n_chunks % depth == 0 and n_chunks >= 2 * depth

    mesh = plsc.VectorSubcoreMesh(core_axis_name="c", subcore_axis_name="s")

    @functools.partial(
        pl.kernel,
        mesh=mesh,
        out_type=jax.ShapeDtypeStruct((B, D), jnp.float32),
        scratch_types=[
            pltpu.VMEM((b_per_w,), jnp.int32),
        ]
        + [pltpu.VMEM((chunk, D), jnp.float32) for _ in range(depth)]
        + [pltpu.SemaphoreType.DMA for _ in range(depth)],
    )
    def gather_kernel(idx_hbm, table_hbm, out_hbm, idx_v, *rest):
        bufs = rest[:depth]
        sems = rest[depth:]
        wid = lax.axis_index("s") * NC + lax.axis_index("c")
        base = wid * b_per_w
        pltpu.sync_copy(idx_hbm.at[pl.ds(base, b_per_w)], idx_v)

        def start(c, b):
            pltpu.async_copy(
                table_hbm.at[idx_v.at[pl.ds(c * chunk, chunk)]], bufs[b], sems[b]
            )

        def finish(c, b):
            pltpu.make_async_copy(
                table_hbm.at[idx_v.at[pl.ds(c * chunk, chunk)]], bufs[b], sems[b]
            ).wait()
            pltpu.sync_copy(bufs[b], out_hbm.at[pl.ds(base + c * chunk, chunk)])

        for b in range(depth):
            start(b, b)

        def body(g, _):
            for b in range(depth):
                finish(g + b, b)
                start(g + b + depth, b)
            return 0

        lax.fori_loop(0, (n_chunks - depth) // depth, lambda i, c: body(i * depth, c), 0)
        for b in range(depth):
            finish(n_chunks - depth + b, b)

    return gather_kernel


def kernel(tokens, masks, table, pe):
    B, S = tokens.shape
    idx = tokens.reshape(-1).astype(jnp.int32)
    gather = _make_gather(B * S, table.shape[0], table.shape[1])
    x = gather(idx, table).reshape(B, S, D_MODEL)
    pos_embed = pe[:S][None, :, :]
    return (x, masks, pos_embed)


# speedup vs baseline: 2.2562x; 1.7633x over previous
import functools

import jax
import jax.numpy as jnp
from jax import lax
from jax.experimental import pallas as pl
from jax.experimental.pallas import tpu as pltpu
from jax.experimental.pallas import tpu_sc as plsc

D_MODEL = 128


@functools.lru_cache(maxsize=None)
def _make_gather(NB, S, V, D):
    info = plsc.get_sparse_core_info()
    NC, NS = info.num_cores, info.num_subcores
    NW = NC * NS
    B = NB * S
    assert B % NW == 0
    b_per_w = B // NW
    cb = 4
    chunk = cb * S
    depth = 4
    assert b_per_w % chunk == 0 and chunk % 8 == 0
    n_chunks = b_per_w // chunk
    nb_per_w = b_per_w // S
    assert n_chunks % depth == 0 and n_chunks >= 2 * depth

    mesh = plsc.VectorSubcoreMesh(core_axis_name="c", subcore_axis_name="s")

    @functools.partial(
        pl.kernel,
        mesh=mesh,
        out_type=jax.ShapeDtypeStruct((NB, S, D), jnp.float32),
        scratch_types=[
            pltpu.VMEM((b_per_w,), jnp.int32),
        ]
        + [pltpu.VMEM((chunk, D), jnp.float32) for _ in range(depth)]
        + [pltpu.SemaphoreType.DMA for _ in range(depth)],
    )
    def gather_kernel(idx_hbm, table_hbm, out_hbm, idx_v, *rest):
        bufs = rest[:depth]
        sems = rest[depth:]
        wid = lax.axis_index("s") * NC + lax.axis_index("c")
        base = wid * b_per_w
        nb_base = wid * nb_per_w
        pltpu.sync_copy(idx_hbm.at[pl.ds(base, b_per_w)], idx_v)

        def start(c, b):
            pltpu.async_copy(
                table_hbm.at[idx_v.at[pl.ds(c * chunk, chunk)]], bufs[b], sems[b]
            )

        def finish(c, b):
            pltpu.make_async_copy(
                table_hbm.at[idx_v.at[pl.ds(c * chunk, chunk)]], bufs[b], sems[b]
            ).wait()
            for j in range(cb):
                pltpu.sync_copy(
                    bufs[b].at[pl.ds(j * S, S)], out_hbm.at[nb_base + c * cb + j]
                )

        for b in range(depth):
            start(b, b)

        def body(g, _):
            for b in range(depth):
                finish(g + b, b)
                start(g + b + depth, b)
            return 0

        lax.fori_loop(0, (n_chunks - depth) // depth, lambda i, c: body(i * depth, c), 0)
        for b in range(depth):
            finish(n_chunks - depth + b, b)

    return gather_kernel


def kernel(tokens, masks, table, pe):
    NB, S = tokens.shape
    idx = tokens.reshape(-1).astype(jnp.int32)
    gather = _make_gather(NB, S, table.shape[0], table.shape[1])
    x = gather(idx, table)
    pos_embed = pe[:S][None, :, :]
    return (x, masks, pos_embed)
